# baseline (speedup 1.0000x reference)
import jax
import jax.numpy as jnp
from jax import lax
from jax.experimental import pallas as pl
from jax.experimental.pallas import tpu as pltpu
from jax.experimental.pallas import tpu_sc as plsc

_EMBED = 1024
_HEADS = 16
_HD = _EMBED // _HEADS
_CS = 8
_K = 2 ** _CS
_SCALE = _HD ** -0.5

_NC = 2
_NS = 16
_GRP = 4
_GW = _EMBED // _GRP
_CHUNK = 32
_SEG = 1024


def _codes_body(k_ref, wc_ref, bc_ref, loc_ref):
    i32 = jnp.int32
    S = k_ref.shape[0]
    code = jax.lax.dot_general(k_ref[...], wc_ref[...], (((1,), (1,)), ((), ())),
                               preferred_element_type=jnp.float32)
    code = code + bc_ref[...]
    bits = (code >= 0.0).astype(i32)
    jj = jax.lax.broadcasted_iota(i32, (S, _CS), 1)
    pw = jax.lax.shift_left(jnp.ones((S, _CS), i32), (_CS - 1) - jj)
    loc_ref[...] = jnp.sum(bits * pw, axis=1, keepdims=True)


def _sc_body(v_hbm, loc_hbm,
             codv_hbm, cnt_hbm, acc, cnt, vbuf, vbuf2, sloc, smloc,
             sem0, sem1):
    c = lax.axis_index("c")
    s = lax.axis_index("s")
    wid = c * _NS + s
    b = wid // _GRP
    g = wid % _GRP

    nch = _SEG // _CHUNK
    vbufs = (vbuf, vbuf2)
    sems = (sem0, sem1)

    def _start(ch, buf, sem):
        pltpu.make_async_copy(
            v_hbm.at[pl.ds(b * _SEG + ch * _CHUNK, _CHUNK),
                     pl.ds(g * _GW, _GW)], buf, sem).start()

    _start(0, vbuf, sem0)
    _start(1, vbuf2, sem1)

    @pl.when(g == 0)
    def _():
        pltpu.sync_copy(loc_hbm.at[pl.ds(b * _SEG, _SEG)], sloc.at[s // _GRP])

    z16f = jnp.zeros((16,), jnp.float32)

    def zero_body(r, carry):
        for u in range(_GW // 16):
            acc[r, pl.ds(u * 16, 16)] = z16f
        cnt[r, pl.ds(0, 16)] = z16f
        return carry

    lax.fori_loop(0, _K, zero_body, 0)

    plsc.subcore_barrier()
    pltpu.sync_copy(sloc.at[s // _GRP], smloc)

    def chunk_pair(j, carry):
        for i in range(2):
            ch = 2 * j + i
            buf, sem = vbufs[i], sems[i]
            pltpu.make_async_copy(
                v_hbm.at[pl.ds(b * _SEG + ch * _CHUNK, _CHUNK),
                         pl.ds(g * _GW, _GW)], buf, sem).wait()

            nu = _GW // 16

            def tok_body(t, c2):
                base = ch * _CHUNK + t * 8
                rows = [smloc[base + tt] for tt in range(8)]
                vals = [buf[t * 8, pl.ds(u * 16, 16)] for u in range(nu)]
                for tt in range(8):
                    nxt = [None] * nu
                    for u in range(nu):
                        if tt < 7:
                            nxt[u] = buf[t * 8 + tt + 1, pl.ds(u * 16, 16)]
                        plsc.addupdate(acc.at[rows[tt], pl.ds(u * 16, 16)],
                                       vals[u])
                    vals = nxt
                return c2

            lax.fori_loop(0, _CHUNK // 8, tok_body, 0)

            @pl.when(ch + 2 < nch)
            def _():
                _start(ch + 2, buf, sem)
        return carry

    lax.fori_loop(0, nch // 2, chunk_pair, 0)

    ones16 = jnp.ones((16,), jnp.float32)
    cseg = _SEG // _GRP

    def cnt_body(t, carry):
        row = smloc[g * cseg + t]
        plsc.addupdate(cnt.at[row], ones16)
        return carry

    lax.fori_loop(0, cseg, cnt_body, 0)
    pltpu.sync_copy(cnt, cnt_hbm.at[b, g])

    pltpu.sync_copy(acc, codv_hbm.at[wid])


def _attn_body(q_ref, codv_ref, cnt_ref, cb_ref, o_ref):
    f32 = jnp.float32
    i32 = jnp.int32
    codv = codv_ref[...]
    cnt4 = jnp.sum(cnt_ref[0], axis=0)
    cntc = jnp.sum(cnt4, axis=1, keepdims=True) * (1.0 / 16.0)
    ii = jax.lax.broadcasted_iota(i32, (_K, 2 * _CS), 0)
    jj = jax.lax.broadcasted_iota(i32, (_K, 2 * _CS), 1)
    sh = jnp.where(jj < _CS, (_CS - 1) - jj, (2 * _CS - 1) - jj)
    bit = jax.lax.shift_right_logical(ii, sh) & 1
    sel = jnp.where(jj < _CS, bit, 1 - bit).astype(f32)
    codk = jax.lax.dot_general(sel, cb_ref[...], (((1,), (0,)), ((), ())),
                               preferred_element_type=f32)
    neg = jnp.where(cntc > 0.0, 0.0, -1e30)
    qb = (q_ref[...] * _SCALE).astype(jnp.bfloat16)
    codk16 = codk.astype(jnp.bfloat16)
    hpg = _GW // _HD
    for h in range(_HEADS):
        sl = slice(h * _HD, (h + 1) * _HD)
        logitsT = jax.lax.dot_general(codk16[:, sl], qb[:, sl],
                                      (((1,), (1,)), ((), ())),
                                      preferred_element_type=f32)
        eT = jnp.exp(logitsT + neg)
        vh = codv[h // hpg, :, (h % hpg) * _HD:(h % hpg + 1) * _HD]
        den = jnp.sum(eT * cntc, axis=0, keepdims=True)
        eTn = eT * (1.0 / den)
        o_ref[:, sl] = jax.lax.dot_general(
            eTn.astype(jnp.bfloat16), vh.astype(jnp.bfloat16),
            (((0,), (0,)), ((), ())), preferred_element_type=f32)


def kernel(q, k, v, Wc, bc, codebook, lengths, inv_lengths):
    L = q.shape[0]
    B = len(lengths)
    seg = L // B
    bc2 = bc.reshape(1, _CS)
    blk = lambda b: (b, 0)
    fixed = lambda b: (0, 0)

    loc = pl.pallas_call(
        _codes_body,
        grid=(B,),
        in_specs=[
            pl.BlockSpec((seg, _EMBED), blk),
            pl.BlockSpec((_CS, _EMBED), fixed),
            pl.BlockSpec((1, _CS), fixed),
        ],
        out_specs=pl.BlockSpec((seg, 1), blk),
        out_shape=jax.ShapeDtypeStruct((L, 1), jnp.int32),
    )(k, Wc, bc2)
    loc1 = loc.reshape(L)

    mesh = plsc.VectorSubcoreMesh(core_axis_name="c", subcore_axis_name="s")
    sc_scatter = pl.kernel(
        _sc_body,
        out_type=[
            jax.ShapeDtypeStruct((_NC * _NS, _K, _GW), jnp.float32),
            jax.ShapeDtypeStruct((B, _GRP, _K, 16), jnp.float32),
        ],
        mesh=mesh,
        compiler_params=pltpu.CompilerParams(needs_layout_passes=False),
        scratch_types=[
            pltpu.VMEM((_K, _GW), jnp.float32),
            pltpu.VMEM((_K, 16), jnp.float32),
            pltpu.VMEM((_CHUNK, _GW), jnp.float32),
            pltpu.VMEM((_CHUNK, _GW), jnp.float32),
            pltpu.VMEM_SHARED((_NS // _GRP, _SEG), jnp.int32),
            pltpu.SMEM((_SEG,), jnp.int32),
            pltpu.SemaphoreType.DMA,
            pltpu.SemaphoreType.DMA,
        ],
    )
    codv, cnt = sc_scatter(v, loc1)

    out = pl.pallas_call(
        _attn_body,
        grid=(B,),
        in_specs=[
            pl.BlockSpec((seg, _EMBED), blk),
            pl.BlockSpec((_GRP, _K, _GW), lambda b: (b, 0, 0)),
            pl.BlockSpec((1, _GRP, _K, 16), lambda b: (b, 0, 0, 0)),
            pl.BlockSpec((2 * _CS, _EMBED), fixed),
        ],
        out_specs=pl.BlockSpec((seg, _EMBED), blk),
        out_shape=jax.ShapeDtypeStruct((L, _EMBED), jnp.float32),
    )(q, codv, cnt, codebook)
    return out


# speedup vs baseline: 1.0071x; 1.0071x over previous
import jax
import jax.numpy as jnp
from jax import lax
from jax.experimental import pallas as pl
from jax.experimental.pallas import tpu as pltpu
from jax.experimental.pallas import tpu_sc as plsc

_EMBED = 1024
_HEADS = 16
_HD = _EMBED // _HEADS
_CS = 8
_K = 2 ** _CS
_SCALE = _HD ** -0.5

_NC = 2
_NS = 16
_GRP = 4
_GW = _EMBED // _GRP
_CHUNK = 32
_SEG = 1024


def _codes_body(k_ref, wc_ref, bc_ref, loc_ref):
    i32 = jnp.int32
    S = k_ref.shape[0]
    code = jax.lax.dot_general(k_ref[...], wc_ref[...], (((1,), (1,)), ((), ())),
                               preferred_element_type=jnp.float32)
    code = code + bc_ref[...]
    bits = (code >= 0.0).astype(i32)
    jj = jax.lax.broadcasted_iota(i32, (S, _CS), 1)
    pw = jax.lax.shift_left(jnp.ones((S, _CS), i32), (_CS - 1) - jj)
    loc_ref[...] = jnp.sum(bits * pw, axis=1, keepdims=True)


def _sc_body(v_hbm, loc_hbm,
             codv_hbm, cnt_hbm, acc, cnt, vbuf, vbuf2, sloc, smloc,
             sem0, sem1):
    c = lax.axis_index("c")
    s = lax.axis_index("s")
    wid = c * _NS + s
    b = wid // _GRP
    g = wid % _GRP

    nch = _SEG // _CHUNK
    vbufs = (vbuf, vbuf2)
    sems = (sem0, sem1)

    def _start(ch, buf, sem):
        pltpu.make_async_copy(
            v_hbm.at[pl.ds(b * _SEG + ch * _CHUNK, _CHUNK),
                     pl.ds(g * _GW, _GW)], buf, sem).start()

    _start(0, vbuf, sem0)
    _start(1, vbuf2, sem1)

    @pl.when(g == 0)
    def _():
        pltpu.sync_copy(loc_hbm.at[pl.ds(b * _SEG, _SEG)], sloc.at[s // _GRP])

    z16f = jnp.zeros((16,), jnp.float32)

    def zero_body(r, carry):
        for u in range(_GW // 16):
            acc[r, pl.ds(u * 16, 16)] = z16f
        cnt[r, pl.ds(0, 16)] = z16f
        return carry

    lax.fori_loop(0, _K, zero_body, 0)

    plsc.subcore_barrier()
    pltpu.sync_copy(sloc.at[s // _GRP], smloc)

    def chunk_pair(j, carry):
        for i in range(2):
            ch = 2 * j + i
            buf, sem = vbufs[i], sems[i]
            pltpu.make_async_copy(
                v_hbm.at[pl.ds(b * _SEG + ch * _CHUNK, _CHUNK),
                         pl.ds(g * _GW, _GW)], buf, sem).wait()

            nu = _GW // 16

            def tok_body(t, c2):
                base = ch * _CHUNK + t * 8
                rows = [smloc[base + tt] for tt in range(8)]
                vals = [buf[t * 8, pl.ds(u * 16, 16)] for u in range(nu)]
                for tt in range(8):
                    nxt = [None] * nu
                    for u in range(nu):
                        if tt < 7:
                            nxt[u] = buf[t * 8 + tt + 1, pl.ds(u * 16, 16)]
                        plsc.addupdate(acc.at[rows[tt], pl.ds(u * 16, 16)],
                                       vals[u])
                    vals = nxt
                return c2

            lax.fori_loop(0, _CHUNK // 8, tok_body, 0)

            @pl.when(ch + 2 < nch)
            def _():
                _start(ch + 2, buf, sem)
        return carry

    lax.fori_loop(0, nch // 2, chunk_pair, 0)

    acc_out = pltpu.make_async_copy(acc, codv_hbm.at[wid], sem0)
    acc_out.start()

    ones16 = jnp.ones((16,), jnp.float32)
    cseg = _SEG // _GRP

    def cnt_body(t, carry):
        row = smloc[g * cseg + t]
        plsc.addupdate(cnt.at[row], ones16)
        return carry

    lax.fori_loop(0, cseg, cnt_body, 0)
    pltpu.sync_copy(cnt, cnt_hbm.at[b, g])
    acc_out.wait()


def _attn_body(q_ref, codv_ref, cnt_ref, cb_ref, o_ref):
    f32 = jnp.float32
    i32 = jnp.int32
    codv = codv_ref[...]
    cnt4 = jnp.sum(cnt_ref[0], axis=0)
    cntc = jnp.sum(cnt4, axis=1, keepdims=True) * (1.0 / 16.0)
    ii = jax.lax.broadcasted_iota(i32, (_K, 2 * _CS), 0)
    jj = jax.lax.broadcasted_iota(i32, (_K, 2 * _CS), 1)
    sh = jnp.where(jj < _CS, (_CS - 1) - jj, (2 * _CS - 1) - jj)
    bit = jax.lax.shift_right_logical(ii, sh) & 1
    sel = jnp.where(jj < _CS, bit, 1 - bit).astype(f32)
    codk = jax.lax.dot_general(sel, cb_ref[...], (((1,), (0,)), ((), ())),
                               preferred_element_type=f32)
    neg = jnp.where(cntc > 0.0, 0.0, -1e30)
    qb = (q_ref[...] * _SCALE).astype(jnp.bfloat16)
    codk16 = codk.astype(jnp.bfloat16)
    hpg = _GW // _HD
    for h in range(_HEADS):
        sl = slice(h * _HD, (h + 1) * _HD)
        logitsT = jax.lax.dot_general(codk16[:, sl], qb[:, sl],
                                      (((1,), (1,)), ((), ())),
                                      preferred_element_type=f32)
        eT = jnp.exp(logitsT + neg)
        vh = codv[h // hpg, :, (h % hpg) * _HD:(h % hpg + 1) * _HD]
        den = jnp.sum(eT * cntc, axis=0, keepdims=True)
        eTn = eT * (1.0 / den)
        o_ref[:, sl] = jax.lax.dot_general(
            eTn.astype(jnp.bfloat16), vh.astype(jnp.bfloat16),
            (((0,), (0,)), ((), ())), preferred_element_type=f32)


def kernel(q, k, v, Wc, bc, codebook, lengths, inv_lengths):
    L = q.shape[0]
    B = len(lengths)
    seg = L // B
    bc2 = bc.reshape(1, _CS)
    blk = lambda b: (b, 0)
    fixed = lambda b: (0, 0)

    loc = pl.pallas_call(
        _codes_body,
        grid=(B,),
        in_specs=[
            pl.BlockSpec((seg, _EMBED), blk),
            pl.BlockSpec((_CS, _EMBED), fixed),
            pl.BlockSpec((1, _CS), fixed),
        ],
        out_specs=pl.BlockSpec((seg, 1), blk),
        out_shape=jax.ShapeDtypeStruct((L, 1), jnp.int32),
    )(k, Wc, bc2)
    loc1 = loc.reshape(L)

    mesh = plsc.VectorSubcoreMesh(core_axis_name="c", subcore_axis_name="s")
    sc_scatter = pl.kernel(
        _sc_body,
        out_type=[
            jax.ShapeDtypeStruct((_NC * _NS, _K, _GW), jnp.float32),
            jax.ShapeDtypeStruct((B, _GRP, _K, 16), jnp.float32),
        ],
        mesh=mesh,
        compiler_params=pltpu.CompilerParams(needs_layout_passes=False),
        scratch_types=[
            pltpu.VMEM((_K, _GW), jnp.float32),
            pltpu.VMEM((_K, 16), jnp.float32),
            pltpu.VMEM((_CHUNK, _GW), jnp.float32),
            pltpu.VMEM((_CHUNK, _GW), jnp.float32),
            pltpu.VMEM_SHARED((_NS // _GRP, _SEG), jnp.int32),
            pltpu.SMEM((_SEG,), jnp.int32),
            pltpu.SemaphoreType.DMA,
            pltpu.SemaphoreType.DMA,
        ],
    )
    codv, cnt = sc_scatter(v, loc1)

    out = pl.pallas_call(
        _attn_body,
        grid=(B,),
        in_specs=[
            pl.BlockSpec((seg, _EMBED), blk),
            pl.BlockSpec((_GRP, _K, _GW), lambda b: (b, 0, 0)),
            pl.BlockSpec((1, _GRP, _K, 16), lambda b: (b, 0, 0, 0)),
            pl.BlockSpec((2 * _CS, _EMBED), fixed),
        ],
        out_specs=pl.BlockSpec((seg, _EMBED), blk),
        out_shape=jax.ShapeDtypeStruct((L, _EMBED), jnp.float32),
    )(q, codv, cnt, codebook)
    return out


# speedup vs baseline: 1.0209x; 1.0137x over previous
import jax
import jax.numpy as jnp
from jax import lax
from jax.experimental import pallas as pl
from jax.experimental.pallas import tpu as pltpu
from jax.experimental.pallas import tpu_sc as plsc

_EMBED = 1024
_HEADS = 16
_HD = _EMBED // _HEADS
_CS = 8
_K = 2 ** _CS
_SCALE = _HD ** -0.5

_NC = 2
_NS = 16
_GRP = 4
_GW = _EMBED // _GRP
_CHUNK = 32
_SEG = 1024


def _codes_body(k_ref, wc_ref, bc_ref, loc_ref):
    i32 = jnp.int32
    S = k_ref.shape[0]
    code = jax.lax.dot_general(k_ref[...], wc_ref[...], (((1,), (1,)), ((), ())),
                               preferred_element_type=jnp.float32)
    code = code + bc_ref[...]
    bits = (code >= 0.0).astype(i32)
    jj = jax.lax.broadcasted_iota(i32, (S, _CS), 1)
    pw = jax.lax.shift_left(jnp.ones((S, _CS), i32), (_CS - 1) - jj)
    loc_ref[...] = jnp.sum(bits * pw, axis=1, keepdims=True)


def _sc_body(v_hbm, loc_hbm,
             codv_hbm, cnt_hbm, acc, cnt, vbuf, vbuf2, sloc, smloc,
             sem0, sem1):
    c = lax.axis_index("c")
    s = lax.axis_index("s")
    wid = c * _NS + s
    b = wid // _GRP
    g = wid % _GRP

    nch = _SEG // _CHUNK
    vbufs = (vbuf, vbuf2)
    sems = (sem0, sem1)

    def _start(ch, buf, sem):
        pltpu.make_async_copy(
            v_hbm.at[pl.ds(b * _SEG + ch * _CHUNK, _CHUNK),
                     pl.ds(g * _GW, _GW)], buf, sem).start()

    _start(0, vbuf, sem0)
    _start(1, vbuf2, sem1)

    @pl.when(g == 0)
    def _():
        pltpu.sync_copy(loc_hbm.at[pl.ds(b * _SEG, _SEG)], sloc.at[s // _GRP])

    z16f = jnp.zeros((16,), jnp.float32)

    def zero_body(r, carry):
        for u in range(_GW // 16):
            acc[r, pl.ds(u * 16, 16)] = z16f
        cnt[r, pl.ds(0, 16)] = z16f
        return carry

    lax.fori_loop(0, _K, zero_body, 0)

    plsc.subcore_barrier()
    pltpu.sync_copy(sloc.at[s // _GRP], smloc)

    def chunk_pair(j, carry):
        for i in range(2):
            ch = 2 * j + i
            buf, sem = vbufs[i], sems[i]
            pltpu.make_async_copy(
                v_hbm.at[pl.ds(b * _SEG + ch * _CHUNK, _CHUNK),
                         pl.ds(g * _GW, _GW)], buf, sem).wait()

            nu = _GW // 16

            def tok_body(t, c2):
                base = ch * _CHUNK + t * 8
                rows = [smloc[base + tt] for tt in range(8)]
                vals = [buf[t * 8, pl.ds(u * 16, 16)] for u in range(nu)]
                for tt in range(8):
                    nxt = [None] * nu
                    for u in range(nu):
                        if tt < 7:
                            nxt[u] = buf[t * 8 + tt + 1, pl.ds(u * 16, 16)]
                        plsc.addupdate(acc.at[rows[tt], pl.ds(u * 16, 16)],
                                       vals[u])
                    vals = nxt
                return c2

            lax.fori_loop(0, _CHUNK // 8, tok_body, 0)

            @pl.when(ch + 2 < nch)
            def _():
                _start(ch + 2, buf, sem)
        return carry

    lax.fori_loop(0, nch // 2, chunk_pair, 0)

    acc_out = pltpu.make_async_copy(acc, codv_hbm.at[wid], sem0)
    acc_out.start()

    ones16 = jnp.ones((16,), jnp.float32)
    cseg = _SEG // _GRP

    def cnt_body(t, carry):
        row = smloc[g * cseg + t]
        plsc.addupdate(cnt.at[row], ones16)
        return carry

    lax.fori_loop(0, cseg, cnt_body, 0)
    pltpu.sync_copy(cnt, cnt_hbm.at[b, g])
    acc_out.wait()


def _attn_body(q_ref, codv_ref, cnt_ref, cb_ref, o_ref):
    f32 = jnp.float32
    i32 = jnp.int32
    codv = codv_ref[...]
    cnt4 = jnp.sum(cnt_ref[0], axis=0)
    cntc = jnp.sum(cnt4, axis=1, keepdims=True) * (1.0 / 16.0)
    ii = jax.lax.broadcasted_iota(i32, (_K, 2 * _CS), 0)
    jj = jax.lax.broadcasted_iota(i32, (_K, 2 * _CS), 1)
    sh = jnp.where(jj < _CS, (_CS - 1) - jj, (2 * _CS - 1) - jj)
    bit = jax.lax.shift_right_logical(ii, sh) & 1
    sel = jnp.where(jj < _CS, bit, 1 - bit).astype(f32)
    codk = jax.lax.dot_general(sel, cb_ref[...], (((1,), (0,)), ((), ())),
                               preferred_element_type=f32)
    qb = (q_ref[...] * _SCALE).astype(jnp.bfloat16)
    codk16 = codk.astype(jnp.bfloat16)
    hpg = _GW // _HD
    for h in range(_HEADS):
        sl = slice(h * _HD, (h + 1) * _HD)
        logitsT = jax.lax.dot_general(codk16[:, sl], qb[:, sl],
                                      (((1,), (1,)), ((), ())),
                                      preferred_element_type=f32)
        eT = jnp.exp(logitsT)
        vh = codv[h // hpg, :, (h % hpg) * _HD:(h % hpg + 1) * _HD]
        den = jnp.sum(eT * cntc, axis=0, keepdims=True)
        eTn = eT * (1.0 / den)
        o_ref[:, sl] = jax.lax.dot_general(
            eTn.astype(jnp.bfloat16), vh.astype(jnp.bfloat16),
            (((0,), (0,)), ((), ())), preferred_element_type=f32)


def kernel(q, k, v, Wc, bc, codebook, lengths, inv_lengths):
    L = q.shape[0]
    B = len(lengths)
    seg = L // B
    bc2 = bc.reshape(1, _CS)
    blk = lambda b: (b, 0)
    fixed = lambda b: (0, 0)

    loc = pl.pallas_call(
        _codes_body,
        grid=(B,),
        in_specs=[
            pl.BlockSpec((seg, _EMBED), blk),
            pl.BlockSpec((_CS, _EMBED), fixed),
            pl.BlockSpec((1, _CS), fixed),
        ],
        out_specs=pl.BlockSpec((seg, 1), blk),
        out_shape=jax.ShapeDtypeStruct((L, 1), jnp.int32),
    )(k, Wc, bc2)
    loc1 = loc.reshape(L)

    mesh = plsc.VectorSubcoreMesh(core_axis_name="c", subcore_axis_name="s")
    sc_scatter = pl.kernel(
        _sc_body,
        out_type=[
            jax.ShapeDtypeStruct((_NC * _NS, _K, _GW), jnp.float32),
            jax.ShapeDtypeStruct((B, _GRP, _K, 16), jnp.float32),
        ],
        mesh=mesh,
        compiler_params=pltpu.CompilerParams(needs_layout_passes=False),
        scratch_types=[
            pltpu.VMEM((_K, _GW), jnp.float32),
            pltpu.VMEM((_K, 16), jnp.float32),
            pltpu.VMEM((_CHUNK, _GW), jnp.float32),
            pltpu.VMEM((_CHUNK, _GW), jnp.float32),
            pltpu.VMEM_SHARED((_NS // _GRP, _SEG), jnp.int32),
            pltpu.SMEM((_SEG,), jnp.int32),
            pltpu.SemaphoreType.DMA,
            pltpu.SemaphoreType.DMA,
        ],
    )
    codv, cnt = sc_scatter(v, loc1)

    out = pl.pallas_call(
        _attn_body,
        grid=(B,),
        in_specs=[
            pl.BlockSpec((seg, _EMBED), blk),
            pl.BlockSpec((_GRP, _K, _GW), lambda b: (b, 0, 0)),
            pl.BlockSpec((1, _GRP, _K, 16), lambda b: (b, 0, 0, 0)),
            pl.BlockSpec((2 * _CS, _EMBED), fixed),
        ],
        out_specs=pl.BlockSpec((seg, _EMBED), blk),
        out_shape=jax.ShapeDtypeStruct((L, _EMBED), jnp.float32),
    )(q, codv, cnt, codebook)
    return out
